# baseline (speedup 1.0000x reference)
import functools

import jax
import jax.numpy as jnp
from jax import lax
from jax.experimental import pallas as pl
from jax.experimental.pallas import tpu as pltpu
from jax.experimental.pallas import tpu_sc as plsc

_TAU = 0.07
_EPS = 1e-12


def _gather_rows_sc(table, ids):
    v_rows, d_dim = table.shape
    batch = ids.shape[0]
    info = plsc.get_sparse_core_info()
    num_workers = info.num_cores * info.num_subcores
    b_per_w = batch // num_workers
    mesh = plsc.VectorSubcoreMesh(core_axis_name="c", subcore_axis_name="s")

    @functools.partial(
        pl.kernel,
        mesh=mesh,
        out_type=jax.ShapeDtypeStruct((batch, d_dim), jnp.float32),
        scratch_types=[
            pltpu.VMEM((b_per_w,), jnp.int32),
            pltpu.VMEM((b_per_w, d_dim), jnp.float32),
            pltpu.SemaphoreType.DMA,
        ],
    )
    def gather_kernel(table_hbm, idx_hbm, out_hbm, idx_v, rows_v, sem):
        wid = lax.axis_index("s") * info.num_cores + lax.axis_index("c")
        base = wid * b_per_w
        pltpu.sync_copy(idx_hbm.at[pl.ds(base, b_per_w)], idx_v)
        pltpu.async_copy(table_hbm.at[idx_v], rows_v, sem).wait()
        pltpu.sync_copy(rows_v, out_hbm.at[pl.ds(base, b_per_w)])

    return gather_kernel(table, ids)


def _simrows_softmax_tc(g_raw, dictionary, tile_b):
    batch, d_dim = g_raw.shape
    k_atoms = dictionary.shape[1]

    def body(g_ref, d_ref, o_ref):
        g = g_ref[...]
        d = d_ref[...]
        g_norm = jnp.sqrt(jnp.sum(g * g, axis=1, keepdims=True))
        gs = g * (1.0 / (jnp.maximum(g_norm, _EPS) * _TAU))
        c_norm = jnp.sqrt(jnp.sum(d * d, axis=0, keepdims=True))
        ds = d * (1.0 / jnp.maximum(c_norm, _EPS))
        s = lax.dot_general(
            gs, ds, (((1,), (0,)), ((), ())),
            preferred_element_type=jnp.float32,
        )
        e = jnp.exp(s)
        r = 1.0 / jnp.sum(e, axis=1, keepdims=True)
        o_ref[...] = e * r

    return pl.pallas_call(
        body,
        grid=(batch // tile_b,),
        in_specs=[
            pl.BlockSpec((tile_b, d_dim), lambda i: (i, 0)),
            pl.BlockSpec((d_dim, k_atoms), lambda i: (0, 0)),
        ],
        out_specs=pl.BlockSpec((tile_b, k_atoms), lambda i: (i, 0)),
        out_shape=jax.ShapeDtypeStruct((batch, k_atoms), jnp.float32),
        compiler_params=pltpu.CompilerParams(
            dimension_semantics=("parallel",),
        ),
    )(g_raw, dictionary)


def kernel(atom_ids, dictionary):
    flat_ids = atom_ids.reshape(-1)
    table = dictionary.T
    g_raw = _gather_rows_sc(table, flat_ids)
    out = _simrows_softmax_tc(g_raw, dictionary, tile_b=512)
    return out.reshape(atom_ids.shape + (dictionary.shape[1],))


# speedup vs baseline: 1.1425x; 1.1425x over previous
import functools

import jax
import jax.numpy as jnp
from jax import lax
from jax.experimental import pallas as pl
from jax.experimental.pallas import tpu as pltpu
from jax.experimental.pallas import tpu_sc as plsc

_TAU = 0.07
_EPS = 1e-12


def _normalize_to_table_tc(dictionary, tile_k):
    d_dim, k_atoms = dictionary.shape

    def body(d_ref, o_ref):
        d = d_ref[...]
        c_norm = jnp.sqrt(jnp.sum(d * d, axis=0, keepdims=True))
        dn = d * (1.0 / jnp.maximum(c_norm, _EPS))
        o_ref[...] = dn.T

    return pl.pallas_call(
        body,
        grid=(k_atoms // tile_k,),
        in_specs=[pl.BlockSpec((d_dim, tile_k), lambda i: (0, i))],
        out_specs=pl.BlockSpec((tile_k, d_dim), lambda i: (i, 0)),
        out_shape=jax.ShapeDtypeStruct((k_atoms, d_dim), jnp.float32),
        compiler_params=pltpu.CompilerParams(
            dimension_semantics=("parallel",),
        ),
    )(dictionary)


def _gather_rows_sc(table, ids):
    v_rows, d_dim = table.shape
    batch = ids.shape[0]
    info = plsc.get_sparse_core_info()
    num_workers = info.num_cores * info.num_subcores
    b_per_w = batch // num_workers
    mesh = plsc.VectorSubcoreMesh(core_axis_name="c", subcore_axis_name="s")

    @functools.partial(
        pl.kernel,
        mesh=mesh,
        out_type=jax.ShapeDtypeStruct((batch, d_dim), jnp.float32),
        scratch_types=[
            pltpu.VMEM((b_per_w,), jnp.int32),
            pltpu.VMEM((b_per_w, d_dim), jnp.float32),
            pltpu.SemaphoreType.DMA,
        ],
    )
    def gather_kernel(table_hbm, idx_hbm, out_hbm, idx_v, rows_v, sem):
        wid = lax.axis_index("s") * info.num_cores + lax.axis_index("c")
        base = wid * b_per_w
        pltpu.sync_copy(idx_hbm.at[pl.ds(base, b_per_w)], idx_v)
        pltpu.async_copy(table_hbm.at[idx_v], rows_v, sem).wait()
        pltpu.sync_copy(rows_v, out_hbm.at[pl.ds(base, b_per_w)])

    return gather_kernel(table, ids)


def _simrows_softmax_tc(g_unit, table, tile_b):
    batch, d_dim = g_unit.shape
    k_atoms = table.shape[0]

    def body(g_ref, t_ref, o_ref):
        gs = g_ref[...] * (1.0 / _TAU)
        s = lax.dot_general(
            gs, t_ref[...], (((1,), (1,)), ((), ())),
            preferred_element_type=jnp.float32,
        )
        e = jnp.exp(s)
        r = 1.0 / jnp.sum(e, axis=1, keepdims=True)
        o_ref[...] = e * r

    return pl.pallas_call(
        body,
        grid=(batch // tile_b,),
        in_specs=[
            pl.BlockSpec((tile_b, d_dim), lambda i: (i, 0)),
            pl.BlockSpec((k_atoms, d_dim), lambda i: (0, 0)),
        ],
        out_specs=pl.BlockSpec((tile_b, k_atoms), lambda i: (i, 0)),
        out_shape=jax.ShapeDtypeStruct((batch, k_atoms), jnp.float32),
        compiler_params=pltpu.CompilerParams(
            dimension_semantics=("parallel",),
        ),
    )(g_unit, table)


def kernel(atom_ids, dictionary):
    flat_ids = atom_ids.reshape(-1)
    table = _normalize_to_table_tc(dictionary, tile_k=2048)
    g_unit = _gather_rows_sc(table, flat_ids)
    out = _simrows_softmax_tc(g_unit, table, tile_b=512)
    return out.reshape(atom_ids.shape + (dictionary.shape[1],))


# speedup vs baseline: 1.1677x; 1.0221x over previous
import functools

import jax
import jax.numpy as jnp
from jax import lax
from jax.experimental import pallas as pl
from jax.experimental.pallas import tpu as pltpu
from jax.experimental.pallas import tpu_sc as plsc

_TAU = 0.07
_EPS = 1e-12


def _normalize_to_table_tc(dictionary, tile_k):
    d_dim, k_atoms = dictionary.shape

    def body(d_ref, o_ref):
        d = d_ref[...]
        c_norm = jnp.sqrt(jnp.sum(d * d, axis=0, keepdims=True))
        dn = d * (1.0 / jnp.maximum(c_norm, _EPS))
        o_ref[...] = dn.T

    return pl.pallas_call(
        body,
        grid=(k_atoms // tile_k,),
        in_specs=[pl.BlockSpec((d_dim, tile_k), lambda i: (0, i))],
        out_specs=pl.BlockSpec((tile_k, d_dim), lambda i: (i, 0)),
        out_shape=jax.ShapeDtypeStruct((k_atoms, d_dim), jnp.float32),
        compiler_params=pltpu.CompilerParams(
            dimension_semantics=("parallel",),
        ),
    )(dictionary)


def _gather_rows_sc(table, ids):
    v_rows, d_dim = table.shape
    batch = ids.shape[0]
    info = plsc.get_sparse_core_info()
    num_workers = info.num_cores * info.num_subcores
    b_per_w = batch // num_workers
    mesh = plsc.VectorSubcoreMesh(core_axis_name="c", subcore_axis_name="s")

    @functools.partial(
        pl.kernel,
        mesh=mesh,
        out_type=jax.ShapeDtypeStruct((batch, d_dim), jnp.float32),
        scratch_types=[
            pltpu.VMEM((b_per_w,), jnp.int32),
            pltpu.VMEM((b_per_w, d_dim), jnp.float32),
            pltpu.SemaphoreType.DMA,
        ],
    )
    def gather_kernel(table_hbm, idx_hbm, out_hbm, idx_v, rows_v, sem):
        wid = lax.axis_index("s") * info.num_cores + lax.axis_index("c")
        base = wid * b_per_w
        pltpu.sync_copy(idx_hbm.at[pl.ds(base, b_per_w)], idx_v)
        pltpu.async_copy(table_hbm.at[idx_v], rows_v, sem).wait()
        pltpu.sync_copy(rows_v, out_hbm.at[pl.ds(base, b_per_w)])

    return gather_kernel(table, ids)


def _simrows_softmax_tc(g_unit, table, tile_b):
    batch, d_dim = g_unit.shape
    k_atoms = table.shape[0]

    def body(g_ref, t_ref, o_ref):
        gs = g_ref[...] * (1.0 / _TAU)
        s = lax.dot_general(
            gs, t_ref[...], (((1,), (1,)), ((), ())),
            preferred_element_type=jnp.float32,
        )
        e = jnp.exp(s)
        r = 1.0 / jnp.sum(e, axis=1, keepdims=True)
        o_ref[...] = e * r

    return pl.pallas_call(
        body,
        grid=(batch // tile_b,),
        in_specs=[
            pl.BlockSpec((tile_b, d_dim), lambda i: (i, 0)),
            pl.BlockSpec((k_atoms, d_dim), lambda i: (0, 0)),
        ],
        out_specs=pl.BlockSpec((tile_b, k_atoms), lambda i: (i, 0)),
        out_shape=jax.ShapeDtypeStruct((batch, k_atoms), jnp.float32),
        compiler_params=pltpu.CompilerParams(
            dimension_semantics=("parallel",),
        ),
    )(g_unit, table)


def kernel(atom_ids, dictionary):
    flat_ids = atom_ids.reshape(-1)
    table = _normalize_to_table_tc(dictionary, tile_k=4096)
    g_unit = _gather_rows_sc(table, flat_ids)
    out = _simrows_softmax_tc(g_unit, table, tile_b=512)
    return out.reshape(atom_ids.shape + (dictionary.shape[1],))
